# Initial kernel scaffold; baseline (speedup 1.0000x reference)
import functools

import jax
import jax.numpy as jnp
from jax import lax
from jax.experimental import pallas as pl
from jax.experimental.pallas import tpu as pltpu
from jax.experimental.pallas import tpu_sc as plsc

EMBED_DIM = 32
SCALE = float(EMBED_DIM) ** 0.5
LANES = 16

_NC = 2
_NS = 16
_NW = _NC * _NS

CHUNK = 1024


def _make_sc_lookup(batch, dim):
    assert batch % (_NW * CHUNK) == 0
    b_per_w = batch // _NW
    n_chunks = b_per_w // CHUNK
    mesh = plsc.VectorSubcoreMesh(core_axis_name="c", subcore_axis_name="s")

    @functools.partial(
        pl.kernel,
        mesh=mesh,
        out_type=jax.ShapeDtypeStruct((batch, dim), jnp.float32),
        scratch_types=[
            pltpu.VMEM((CHUNK,), jnp.int32),
            pltpu.VMEM((CHUNK, dim), jnp.float32),
            pltpu.SemaphoreType.DMA,
        ],
    )
    def lookup(table_hbm, tokens_hbm, out_hbm, idx_v, rows_v, sem):
        wid = lax.axis_index("s") * _NC + lax.axis_index("c")
        base = wid * b_per_w

        def chunk_body(c, carry):
            off = base + c * CHUNK
            pltpu.sync_copy(tokens_hbm.at[pl.ds(off, CHUNK)], idx_v)
            pltpu.async_copy(table_hbm.at[idx_v], rows_v, sem).wait()

            def scale_body(i, carry2):
                rows_v[i, 0:LANES] = rows_v[i, 0:LANES] * SCALE
                rows_v[i, LANES:2 * LANES] = rows_v[i, LANES:2 * LANES] * SCALE
                return carry2

            lax.fori_loop(0, CHUNK, scale_body, 0, unroll=4)
            pltpu.sync_copy(rows_v, out_hbm.at[pl.ds(off, CHUNK)])
            return carry

        lax.fori_loop(0, n_chunks, chunk_body, 0)

    return lookup


def kernel(tokens, table):
    n_rows, n_cols = tokens.shape
    batch = n_rows * n_cols
    flat = tokens.reshape(batch).astype(jnp.int32)
    out = _make_sc_lookup(batch, EMBED_DIM)(table, flat)
    return out.reshape(n_rows, n_cols, EMBED_DIM)


# speedup vs baseline: 1.3992x; 1.3992x over previous
import functools

import jax
import jax.numpy as jnp
from jax import lax
from jax.experimental import pallas as pl
from jax.experimental.pallas import tpu as pltpu
from jax.experimental.pallas import tpu_sc as plsc

EMBED_DIM = 32
SCALE = float(EMBED_DIM) ** 0.5
LANES = 16

_NC = 2
_NS = 16
_NW = _NC * _NS

CHUNK = 1024


def _make_sc_lookup(batch, dim):
    assert batch % (_NW * CHUNK) == 0
    b_per_w = batch // _NW
    n_chunks = b_per_w // CHUNK
    mesh = plsc.VectorSubcoreMesh(core_axis_name="c", subcore_axis_name="s")

    @functools.partial(
        pl.kernel,
        mesh=mesh,
        out_type=jax.ShapeDtypeStruct((batch, dim), jnp.float32),
        scratch_types=[
            pltpu.VMEM((CHUNK,), jnp.int32),
            pltpu.VMEM((CHUNK, dim), jnp.float32),
            pltpu.SemaphoreType.DMA,
        ],
        compiler_params=pltpu.CompilerParams(use_tc_tiling_on_sc=False),
    )
    def lookup(table_hbm, tokens_hbm, out_hbm, idx_v, rows_v, sem):
        wid = lax.axis_index("s") * _NC + lax.axis_index("c")
        base = wid * b_per_w

        def chunk_body(c, carry):
            off = base + c * CHUNK
            pltpu.sync_copy(tokens_hbm.at[pl.ds(off, CHUNK)], idx_v)
            pltpu.async_copy(table_hbm.at[idx_v], rows_v, sem).wait()

            def scale_body(i, carry2):
                rows_v[i, 0:LANES] = rows_v[i, 0:LANES] * SCALE
                rows_v[i, LANES:2 * LANES] = rows_v[i, LANES:2 * LANES] * SCALE
                return carry2

            lax.fori_loop(0, CHUNK, scale_body, 0, unroll=4)
            pltpu.sync_copy(rows_v, out_hbm.at[pl.ds(off, CHUNK)])
            return carry

        lax.fori_loop(0, n_chunks, chunk_body, 0)

    return lookup


def kernel(tokens, table):
    n_rows, n_cols = tokens.shape
    batch = n_rows * n_cols
    flat = tokens.reshape(batch).astype(jnp.int32)
    out = _make_sc_lookup(batch, EMBED_DIM)(table, flat)
    return out.reshape(n_rows, n_cols, EMBED_DIM)


# speedup vs baseline: 1.4787x; 1.0568x over previous
import functools

import jax
import jax.numpy as jnp
from jax import lax
from jax.experimental import pallas as pl
from jax.experimental.pallas import tpu as pltpu
from jax.experimental.pallas import tpu_sc as plsc

EMBED_DIM = 32
SCALE = float(EMBED_DIM) ** 0.5
LANES = 16

_NC = 2
_NS = 16
_NW = _NC * _NS

CHUNK = 1280


def _make_sc_lookup(batch, dim):
    assert batch % (_NW * CHUNK) == 0
    b_per_w = batch // _NW
    n_chunks = b_per_w // CHUNK
    mesh = plsc.VectorSubcoreMesh(core_axis_name="c", subcore_axis_name="s")

    @functools.partial(
        pl.kernel,
        mesh=mesh,
        out_type=jax.ShapeDtypeStruct((batch, dim), jnp.float32),
        scratch_types=[
            pltpu.VMEM((b_per_w,), jnp.int32),
            pltpu.VMEM((CHUNK, dim), jnp.float32),
            pltpu.VMEM((CHUNK, dim), jnp.float32),
            pltpu.SemaphoreType.DMA,
            pltpu.SemaphoreType.DMA,
            pltpu.SemaphoreType.DMA,
            pltpu.SemaphoreType.DMA,
        ],
        compiler_params=pltpu.CompilerParams(use_tc_tiling_on_sc=False),
    )
    def lookup(table_hbm, tokens_hbm, out_hbm, idx_v, rows0, rows1,
               g0, g1, s0, s1):
        wid = lax.axis_index("s") * _NC + lax.axis_index("c")
        base = wid * b_per_w
        pltpu.sync_copy(tokens_hbm.at[pl.ds(base, b_per_w)], idx_v)

        rows = (rows0, rows1)
        gsem = (g0, g1)
        ssem = (s0, s1)

        def start_gather(c):
            return pltpu.async_copy(
                table_hbm.at[idx_v.at[pl.ds(c * CHUNK, CHUNK)]],
                rows[c % 2], gsem[c % 2])

        def make_scale_body(buf):
            def scale_body(i, carry):
                buf[i, 0:LANES] = buf[i, 0:LANES] * SCALE
                buf[i, LANES:2 * LANES] = buf[i, LANES:2 * LANES] * SCALE
                return carry
            return scale_body

        scale_bodies = (make_scale_body(rows0), make_scale_body(rows1))

        gathers = [start_gather(0)]
        stores = [None, None]
        for c in range(n_chunks):
            b = c % 2
            gathers[c].wait()
            if c + 1 < n_chunks:
                nb = (c + 1) % 2
                if stores[nb] is not None:
                    stores[nb].wait()
                gathers.append(start_gather(c + 1))
            lax.fori_loop(0, CHUNK, scale_bodies[b], 0, unroll=4)
            stores[b] = pltpu.async_copy(
                rows[b], out_hbm.at[pl.ds(base + c * CHUNK, CHUNK)], ssem[b])
        for st in stores:
            if st is not None:
                st.wait()

    return lookup


def kernel(tokens, table):
    n_rows, n_cols = tokens.shape
    batch = n_rows * n_cols
    flat = tokens.reshape(batch).astype(jnp.int32)
    out = _make_sc_lookup(batch, EMBED_DIM)(table, flat)
    return out.reshape(n_rows, n_cols, EMBED_DIM)
